# baseline (speedup 1.0000x reference)
import functools

import jax
import jax.numpy as jnp
from jax import lax
from jax.experimental import pallas as pl
from jax.experimental.pallas import tpu as pltpu
from jax.experimental.pallas import tpu_sc as plsc

N_ENTRIES = 1000000
PARAM_DIM = 64
BATCH = 16384

NC = 2
NS = 16
NW = NC * NS
B_PER_W = BATCH // NW
L = 16
NSEM = 8

_mesh = plsc.VectorSubcoreMesh(core_axis_name="c", subcore_axis_name="s")


@functools.partial(
    pl.kernel,
    out_type=jax.ShapeDtypeStruct((BATCH, PARAM_DIM), jnp.float32),
    mesh=_mesh,
    scratch_types=[
        pltpu.VMEM((B_PER_W,), jnp.int32),
        pltpu.VMEM((B_PER_W, PARAM_DIM), jnp.float32),
    ] + [pltpu.SemaphoreType.DMA] * NSEM,
    compiler_params=pltpu.CompilerParams(needs_layout_passes=False),
)
def _lookup_kernel(x_hbm, table_hbm, out_hbm, idx_v, out_v, *sems):
    wid = lax.axis_index("s") * NC + lax.axis_index("c")
    base = wid * B_PER_W

    pltpu.sync_copy(x_hbm.at[pl.ds(base, B_PER_W)], idx_v)

    def body(g, _):
        vec = idx_v[pl.ds(g * L, L)]
        for k2 in range(L):
            i = vec[k2]
            pltpu.async_copy(
                table_hbm.at[i], out_v.at[g * L + k2], sems[k2 % NSEM],
            )
        return 0

    lax.fori_loop(0, B_PER_W // L, body, 0)
    for q in range(NSEM):
        pltpu.make_async_copy(
            table_hbm.at[pl.ds(0, B_PER_W // NSEM)],
            out_v.reshape(NSEM, B_PER_W // NSEM, PARAM_DIM).at[q],
            sems[q],
        ).wait()

    pltpu.sync_copy(out_v, out_hbm.at[pl.ds(base, B_PER_W)])


def kernel(x, table):
    return _lookup_kernel(x, table)
